# baseline (speedup 1.0000x reference)
import jax
import jax.numpy as jnp
from jax.experimental import pallas as pl
from jax.experimental.pallas import tpu as pltpu

N = 4096
W = 8
BK = 512
K1, K2 = 2048, 1024
_SIGN = -2147483648
_MAXP = 2147483647


def _bf(x):
    return x.astype(jnp.bfloat16)


def _r32(x):
    return x.astype(jnp.bfloat16).astype(jnp.float32)


def _skey(s):
    si = jax.lax.bitcast_convert_type(s, jnp.int32)
    return si ^ ((si >> 31) & jnp.int32(_MAXP))


def _topk_threshold(skey_row, k):
    kf = jnp.float32(k)

    def body(b, prefix):
        cand = prefix | (jnp.int32(1) << (jnp.int32(31) - b))
        cnt = jnp.sum((skey_row >= (cand ^ jnp.int32(_SIGN)))
                      .astype(jnp.float32))
        return jnp.where(cnt >= kf, cand, prefix)

    prefix = jax.lax.fori_loop(0, 32, body, jnp.int32(0))
    t = prefix ^ jnp.int32(_SIGN)

    eq = skey_row == t
    need = kf - jnp.sum((skey_row > t).astype(jnp.float32))
    idx = jax.lax.broadcasted_iota(jnp.int32, skey_row.shape, 1)

    def ibody(b, low):
        cand = low | (jnp.int32(1) << (jnp.int32(12) - b))
        cnt = jnp.sum((eq & (idx < cand)).astype(jnp.float32))
        return jnp.where(cnt < need, cand, low)

    low = jax.lax.fori_loop(0, 13, ibody, jnp.int32(0))
    return t, low + 1


def _topk_mask(skey, t, istar, axis):
    idx = jax.lax.broadcasted_iota(jnp.int32, skey.shape, axis)
    keep = (skey > t) | ((skey == t) & (idx < istar))
    return keep.astype(jnp.float32)


def _degree_kernel(g_ref, d_ref, gbf_ref):
    x = g_ref[...]
    d_ref[...] = jnp.sum(x, axis=1, keepdims=True)
    gbf_ref[...] = _bf(x)


def _anorm_block(g_ref, dblk_ref, drow_ref, i):
    disr = jax.lax.rsqrt(dblk_ref[...] + 2.0)
    disc = jax.lax.rsqrt(drow_ref[...] + 2.0)
    r0 = jax.lax.broadcasted_iota(jnp.int32, (BK, N), 0) + i * BK
    c0 = jax.lax.broadcasted_iota(jnp.int32, (BK, N), 1)
    a = g_ref[...] + jnp.where(r0 == c0, 2.0, 0.0)
    return _bf(a * disr * disc)


def _gcn0_kernel(g_ref, dblk_ref, drow_ref, h_ref, w1t_ref, cst_ref, o_ref):
    i = pl.program_id(0)
    an = _anorm_block(g_ref, dblk_ref, drow_ref, i)
    y = jnp.dot(an, _bf(h_ref[...]), preferred_element_type=jnp.float32)
    o_ref[...] = jax.nn.relu(
        jnp.dot(_bf(y), _bf(w1t_ref[...]),
                preferred_element_type=jnp.float32) + cst_ref[0:1, 0:W])


def _gcnu2_kernel(g_ref, dblk_ref, drow_ref, h4_ref, h1_ref, cst_ref, o_ref):
    i = pl.program_id(0)
    an = _anorm_block(g_ref, dblk_ref, drow_ref, i)
    y = jnp.dot(an, _bf(h4_ref[...]), preferred_element_type=jnp.float32)
    o_ref[...] = jax.nn.relu(
        jnp.dot(_bf(y), _bf(cst_ref[0:W, 0:W]),
                preferred_element_type=jnp.float32)
        + cst_ref[8:9, 0:W]) + h1_ref[...]


def _mid_kernel(g_ref, h1_ref, cst_ref, out_ref, y_ref):
    RB = 1024

    def G(v):
        vq = _bf(v)

        def body(i, carry):
            blk = jnp.dot(g_ref[pl.ds(i * RB, RB), :], vq,
                          preferred_element_type=jnp.float32)
            y_ref[pl.ds(i * RB, RB), :] = blk
            return carry

        jax.lax.fori_loop(0, N // RB, body, 0)
        return y_ref[...]

    b_d2 = cst_ref[1:2, 0:W]
    b_bot = cst_ref[2:3, 0:W]
    b_u1 = cst_ref[3:4, 0:W]
    p1_b = cst_ref[5:6, 0:1]
    p2_b = cst_ref[6:7, 0:1]
    p1w = cst_ref[7:8, 0:W]
    p2w = cst_ref[8:9, 0:W]
    W2m = cst_ref[16:16 + W, 0:W]
    Wbm = cst_ref[24:24 + W, 0:W]
    Wu1m = cst_ref[32:32 + W, 0:W]

    ones0 = (jax.lax.broadcasted_iota(jnp.int32, (N, W), 1) == 0)
    ones0 = ones0.astype(jnp.float32)
    h1 = h1_ref[...]

    proj1 = jnp.sum(_r32(h1) * _r32(p1w), axis=1, keepdims=True) + p1_b
    proj1r = jax.lax.dot_general(_bf(p1w), _bf(h1), (((1,), (1,)), ((), ())),
                                 preferred_element_type=jnp.float32) + p1_b
    T1, I1s = _topk_threshold(_skey(proj1r), K1)
    m1r = _topk_mask(_skey(proj1r), T1, I1s, 1)
    m1 = _topk_mask(_skey(proj1), T1, I1s, 0)
    X1 = m1 * h1 * jax.nn.sigmoid(proj1)

    u = G(m1 * ones0)
    d1 = m1[:, 0:1] * G(u)[:, 0:1] + 2.0
    dis1 = m1 * jax.lax.rsqrt(d1)
    Z1 = dis1 * X1
    Y2 = dis1 * (m1 * G(G(Z1)) + 2.0 * Z1)
    h2 = m1 * jax.nn.relu(
        jnp.dot(_bf(Y2), _bf(W2m), preferred_element_type=jnp.float32)
        + b_d2)

    proj2 = jnp.sum(_r32(h2) * _r32(p2w), axis=1, keepdims=True) + p2_b
    proj2r = jax.lax.dot_general(_bf(p2w), _bf(h2), (((1,), (1,)), ((), ())),
                                 preferred_element_type=jnp.float32) + p2_b
    sk2r = jnp.where(m1r > 0, _skey(proj2r), jnp.int32(_SIGN))
    T2, I2s = _topk_threshold(sk2r, K2)
    sk2 = jnp.where(m1 > 0, _skey(proj2), jnp.int32(_SIGN))
    m2 = _topk_mask(sk2, T2, I2s, 0)
    X2 = m2 * h2 * jax.nn.sigmoid(proj2)

    c = G(m1 * G(G(m2 * ones0)))
    d2 = m2[:, 0:1] * G(c)[:, 0:1] + 2.0
    dis2 = m2 * jax.lax.rsqrt(d2)
    Z2 = dis2 * X2
    Y3 = dis2 * (m2 * G(m1 * G(G(Z2))) + 2.0 * Z2)
    h3 = m2 * jax.nn.relu(
        jnp.dot(_bf(Y3), _bf(Wbm), preferred_element_type=jnp.float32)
        + b_bot)

    Z3 = dis1 * h3
    Y4 = dis1 * (m1 * G(G(Z3)) + 2.0 * Z3)
    out_ref[...] = m1 * jax.nn.relu(
        jnp.dot(_bf(Y4), _bf(Wu1m), preferred_element_type=jnp.float32)
        + b_u1) + h2


def kernel(g, h, W_d1, b_d1, p1_w, p1_b, W_d2, b_d2, p2_w, p2_b,
           W_bot, b_bot, W_u1, b_u1, W_u2, b_u2):
    f32 = jnp.float32
    w1t = jnp.pad(W_d1.T, ((0, 0), (0, W - 3)))

    def pad8(x):
        return jnp.pad(x, ((0, W - x.shape[0]), (0, W - x.shape[1])))

    z13 = jnp.zeros((1, W - 3), f32)
    z17 = jnp.zeros((1, W - 1), f32)
    row = lambda v3: jnp.concatenate([v3[None, :], z13], axis=1)
    rows09 = jnp.concatenate([
        row(b_d1), row(b_d2), row(b_bot), row(b_u1), row(b_u2),
        jnp.concatenate([p1_b[None, :], z17], axis=1),
        jnp.concatenate([p2_b[None, :], z17], axis=1),
        row(p1_w[0]), row(p2_w[0]),
        jnp.zeros((7, W), f32),
    ], axis=0)
    cst = jnp.concatenate([
        rows09, pad8(W_d2.T), pad8(W_bot.T), pad8(W_u1.T),
    ], axis=0)
    cstu = jnp.concatenate([pad8(W_u2.T), row(b_u2)], axis=0)
    cst1 = row(b_d1)

    grid = (N // BK,)
    blk = lambda i: (i, 0)
    whole = lambda i: (0, 0)

    d0, g_bf = pl.pallas_call(
        _degree_kernel,
        grid=grid,
        in_specs=[pl.BlockSpec((BK, N), blk)],
        out_specs=[pl.BlockSpec((BK, 1), blk), pl.BlockSpec((BK, N), blk)],
        out_shape=[jax.ShapeDtypeStruct((N, 1), f32),
                   jax.ShapeDtypeStruct((N, N), jnp.bfloat16)],
    )(g)
    d0r = d0.reshape(1, N)

    h1 = pl.pallas_call(
        _gcn0_kernel,
        grid=grid,
        in_specs=[
            pl.BlockSpec((BK, N), blk),
            pl.BlockSpec((BK, 1), blk),
            pl.BlockSpec((1, N), whole),
            pl.BlockSpec((N, 128), whole),
            pl.BlockSpec((128, W), whole),
            pl.BlockSpec((1, W), whole),
        ],
        out_specs=pl.BlockSpec((BK, W), blk),
        out_shape=jax.ShapeDtypeStruct((N, W), f32),
    )(g, d0, d0r, h, w1t, cst1)

    h4 = pl.pallas_call(
        _mid_kernel,
        out_shape=jax.ShapeDtypeStruct((N, W), f32),
        scratch_shapes=[pltpu.VMEM((N, W), f32)],
    )(g_bf, h1, cst)

    h5 = pl.pallas_call(
        _gcnu2_kernel,
        grid=grid,
        in_specs=[
            pl.BlockSpec((BK, N), blk),
            pl.BlockSpec((BK, 1), blk),
            pl.BlockSpec((1, N), whole),
            pl.BlockSpec((N, W), whole),
            pl.BlockSpec((BK, W), blk),
            pl.BlockSpec((16, W), whole),
        ],
        out_specs=pl.BlockSpec((BK, W), blk),
        out_shape=jax.ShapeDtypeStruct((N, W), f32),
    )(g, d0, d0r, h4, h1, jnp.pad(cstu, ((0, 7), (0, 0))))
    return (h5[:, :3], g)
